# baseline (speedup 1.0000x reference)
import functools

import jax
import jax.numpy as jnp
from jax import lax
from jax.experimental import pallas as pl
from jax.experimental.pallas import tpu as pltpu
from jax.experimental.pallas import tpu_sc as plsc

D = 128
NC, NS, L = 2, 16, 16
NW = NC * NS
CHUNK = 128
GROUPS = CHUNK // L


def _sc_body(vi_idx, vj_idx, node_t, ctx_t, out,
             idx_i, idx_j, vi_rows, vj_rows, out_v, sem_i, sem_j):
    n_chunks = vi_idx.shape[1]
    wid = lax.axis_index("s") * NC + lax.axis_index("c")
    pltpu.sync_copy(vi_idx.at[wid], idx_i)
    pltpu.sync_copy(vj_idx.at[wid], idx_j)

    def chunk_body(c, carry):
        cp_i = pltpu.async_copy(node_t.at[idx_i.at[c]], vi_rows, sem_i)
        cp_j = pltpu.async_copy(ctx_t.at[idx_j.at[c]], vj_rows, sem_j)
        cp_i.wait()
        cp_j.wait()
        for g in range(GROUPS):
            rows = lax.iota(jnp.int32, L) + g * L

            def d_body(t, acc):
                for u in range(8):
                    dd = jnp.full((L,), t * 8 + u, jnp.int32)
                    a = plsc.load_gather(vi_rows, [rows, dd])
                    b = plsc.load_gather(vj_rows, [rows, dd])
                    acc = acc + a * b
                return acc

            acc = lax.fori_loop(0, D // 8, d_body, jnp.zeros((L,), jnp.float32))
            out_v[pl.ds(g * L, L)] = acc
        pltpu.sync_copy(out_v, out.at[wid, c])
        return carry

    lax.fori_loop(0, n_chunks, chunk_body, 0)


def _inner_products(vi_idx, vj_idx, node_t, ctx_t):
    n_chunks = vi_idx.shape[1]
    mesh = plsc.VectorSubcoreMesh(core_axis_name="c", subcore_axis_name="s")
    return pl.kernel(
        _sc_body,
        out_type=jax.ShapeDtypeStruct((NW, n_chunks, CHUNK), jnp.float32),
        mesh=mesh,
        compiler_params=pltpu.CompilerParams(needs_layout_passes=False),
        scratch_types=[
            pltpu.VMEM((n_chunks, CHUNK), jnp.int32),
            pltpu.VMEM((n_chunks, CHUNK), jnp.int32),
            pltpu.VMEM((CHUNK, D), jnp.float32),
            pltpu.VMEM((CHUNK, D), jnp.float32),
            pltpu.VMEM((CHUNK,), jnp.float32),
            pltpu.SemaphoreType.DMA,
            pltpu.SemaphoreType.DMA,
        ],
    )(vi_idx, vj_idx, node_t, ctx_t)


def _tc_loss_body(ip_ref, lab_ref, out_ref):
    x = lab_ref[...] * ip_ref[...]
    ls = jnp.minimum(x, 0.0) - jnp.log1p(jnp.exp(-jnp.abs(x)))
    out_ref[0, 0] = -jnp.sum(ls)


def _tc_loss(ip, labels):
    rows = ip.shape[0]
    return pl.pallas_call(
        _tc_loss_body,
        out_shape=jax.ShapeDtypeStruct((1, 1), jnp.float32),
        in_specs=[
            pl.BlockSpec((rows, D), lambda: (0, 0)),
            pl.BlockSpec((rows, D), lambda: (0, 0)),
        ],
        out_specs=pl.BlockSpec(memory_space=pltpu.SMEM),
    )(ip, labels)


def kernel(v_i, v_j, labels, batch_size, node_embeddings, contextnode_embeddings):
    n_pairs = v_i.shape[0] * v_i.shape[1]
    n_chunks = n_pairs // (NW * CHUNK)
    vi_idx = v_i.reshape(NW, n_chunks, CHUNK).astype(jnp.int32)
    vj_idx = v_j.reshape(NW, n_chunks, CHUNK).astype(jnp.int32)
    ip = _inner_products(vi_idx, vj_idx, node_embeddings, contextnode_embeddings)
    ip2 = ip.reshape(n_pairs // D, D)
    lab2 = labels.reshape(n_pairs // D, D)
    total = _tc_loss(ip2, lab2)[0, 0]
    return total / batch_size


# speedup vs baseline: 3.6312x; 3.6312x over previous
import functools

import jax
import jax.numpy as jnp
from jax import lax
from jax.experimental import pallas as pl
from jax.experimental.pallas import tpu as pltpu
from jax.experimental.pallas import tpu_sc as plsc

D = 128
NC, NS, L = 2, 16, 16
NW = NC * NS
CHUNK = 128
GROUPS = CHUNK // L


def _sc_body(vi_idx, vj_idx, node_t, ctx_t, out,
             idx_i, idx_j, vi_rows, vj_rows, out_v, sem_i, sem_j):
    n_chunks = vi_idx.shape[1]
    wid = lax.axis_index("s") * NC + lax.axis_index("c")
    pltpu.sync_copy(vi_idx.at[wid], idx_i)
    pltpu.sync_copy(vj_idx.at[wid], idx_j)

    def chunk_body(c, carry):
        cp_i = pltpu.async_copy(node_t.at[idx_i.at[c]], vi_rows, sem_i)
        cp_j = pltpu.async_copy(ctx_t.at[idx_j.at[c]], vj_rows, sem_j)
        cp_i.wait()
        cp_j.wait()
        for g in range(GROUPS):
            rows = lax.iota(jnp.int32, L) + g * L

            def d_body(t, carry):
                a0, a1, a2, a3, dd = carry
                accs = [a0, a1, a2, a3]
                for u in range(16):
                    a = plsc.load_gather(vi_rows, [rows, dd])
                    b = plsc.load_gather(vj_rows, [rows, dd])
                    accs[u % 4] = accs[u % 4] + a * b
                    dd = jnp.bitwise_and(dd + 1, D - 1)
                return (accs[0], accs[1], accs[2], accs[3], dd)

            z = jnp.zeros((L,), jnp.float32)
            fin = lax.fori_loop(0, D // 16, d_body,
                                (z, z, z, z, lax.iota(jnp.int32, L)))
            out_v[pl.ds(g * L, L)] = (fin[0] + fin[1]) + (fin[2] + fin[3])
        pltpu.sync_copy(out_v, out.at[wid, c])
        return carry

    lax.fori_loop(0, n_chunks, chunk_body, 0)


def _inner_products(vi_idx, vj_idx, node_t, ctx_t):
    n_chunks = vi_idx.shape[1]
    mesh = plsc.VectorSubcoreMesh(core_axis_name="c", subcore_axis_name="s")
    return pl.kernel(
        _sc_body,
        out_type=jax.ShapeDtypeStruct((NW, n_chunks, CHUNK), jnp.float32),
        mesh=mesh,
        compiler_params=pltpu.CompilerParams(needs_layout_passes=False),
        scratch_types=[
            pltpu.VMEM((n_chunks, CHUNK), jnp.int32),
            pltpu.VMEM((n_chunks, CHUNK), jnp.int32),
            pltpu.VMEM((CHUNK, D), jnp.float32),
            pltpu.VMEM((CHUNK, D), jnp.float32),
            pltpu.VMEM((CHUNK,), jnp.float32),
            pltpu.SemaphoreType.DMA,
            pltpu.SemaphoreType.DMA,
        ],
    )(vi_idx, vj_idx, node_t, ctx_t)


def _tc_loss_body(ip_ref, lab_ref, out_ref):
    x = lab_ref[...] * ip_ref[...]
    ls = jnp.minimum(x, 0.0) - jnp.log1p(jnp.exp(-jnp.abs(x)))
    out_ref[0, 0] = -jnp.sum(ls)


def _tc_loss(ip, labels):
    rows = ip.shape[0]
    return pl.pallas_call(
        _tc_loss_body,
        out_shape=jax.ShapeDtypeStruct((1, 1), jnp.float32),
        in_specs=[
            pl.BlockSpec((rows, D), lambda: (0, 0)),
            pl.BlockSpec((rows, D), lambda: (0, 0)),
        ],
        out_specs=pl.BlockSpec(memory_space=pltpu.SMEM),
    )(ip, labels)


def kernel(v_i, v_j, labels, batch_size, node_embeddings, contextnode_embeddings):
    n_pairs = v_i.shape[0] * v_i.shape[1]
    n_chunks = n_pairs // (NW * CHUNK)
    vi_idx = v_i.reshape(NW, n_chunks, CHUNK).astype(jnp.int32)
    vj_idx = v_j.reshape(NW, n_chunks, CHUNK).astype(jnp.int32)
    ip = _inner_products(vi_idx, vj_idx, node_embeddings, contextnode_embeddings)
    ip2 = ip.reshape(n_pairs // D, D)
    lab2 = labels.reshape(n_pairs // D, D)
    total = _tc_loss(ip2, lab2)[0, 0]
    return total / batch_size


# speedup vs baseline: 4.8510x; 1.3359x over previous
import functools

import jax
import jax.numpy as jnp
from jax import lax
from jax.experimental import pallas as pl
from jax.experimental.pallas import tpu as pltpu
from jax.experimental.pallas import tpu_sc as plsc

D = 128
NC, NS, L = 2, 16, 16
NW = NC * NS
CHUNK = 128
GROUPS = CHUNK // L


def _sc_body(vi_idx, vj_idx, node_t, ctx_t, out,
             idx_i, idx_j, vi_a, vj_a, vi_b, vj_b, out_v, sem_a, sem_b):
    n_chunks = vi_idx.shape[1]
    wid = lax.axis_index("s") * NC + lax.axis_index("c")
    pltpu.sync_copy(vi_idx.at[wid], idx_i)
    pltpu.sync_copy(vj_idx.at[wid], idx_j)

    def start(c, vi_buf, vj_buf, sem):
        pltpu.async_copy(node_t.at[idx_i.at[c]], vi_buf, sem)
        pltpu.async_copy(ctx_t.at[idx_j.at[c]], vj_buf, sem)

    def drain(vi_buf, vj_buf, sem):
        pltpu.make_async_copy(node_t.at[idx_i.at[0]], vi_buf, sem).wait()
        pltpu.make_async_copy(ctx_t.at[idx_j.at[0]], vj_buf, sem).wait()

    def compute(c, vi_rows, vj_rows):
        for g in range(GROUPS):
            rows = lax.iota(jnp.int32, L) + g * L

            def d_body(t, carry):
                a0, a1, a2, a3, dd = carry
                accs = [a0, a1, a2, a3]
                for u in range(16):
                    a = plsc.load_gather(vi_rows, [rows, dd])
                    b = plsc.load_gather(vj_rows, [rows, dd])
                    accs[u % 4] = accs[u % 4] + a * b
                    dd = jnp.bitwise_and(dd + 1, D - 1)
                return (accs[0], accs[1], accs[2], accs[3], dd)

            z = jnp.zeros((L,), jnp.float32)
            fin = lax.fori_loop(0, D // 16, d_body,
                                (z, z, z, z, lax.iota(jnp.int32, L)))
            out_v[pl.ds(g * L, L)] = (fin[0] + fin[1]) + (fin[2] + fin[3])
        pltpu.sync_copy(out_v, out.at[wid, c])

    start(0, vi_a, vj_a, sem_a)

    def super_body(t, carry):
        c0 = 2 * t
        start(c0 + 1, vi_b, vj_b, sem_b)
        drain(vi_a, vj_a, sem_a)
        compute(c0, vi_a, vj_a)

        @pl.when(t + 1 < n_chunks // 2)
        def _():
            start(c0 + 2, vi_a, vj_a, sem_a)

        drain(vi_b, vj_b, sem_b)
        compute(c0 + 1, vi_b, vj_b)
        return carry

    lax.fori_loop(0, n_chunks // 2, super_body, 0)


def _inner_products(vi_idx, vj_idx, node_t, ctx_t):
    n_chunks = vi_idx.shape[1]
    mesh = plsc.VectorSubcoreMesh(core_axis_name="c", subcore_axis_name="s")
    return pl.kernel(
        _sc_body,
        out_type=jax.ShapeDtypeStruct((NW, n_chunks, CHUNK), jnp.float32),
        mesh=mesh,
        compiler_params=pltpu.CompilerParams(needs_layout_passes=False),
        scratch_types=[
            pltpu.VMEM((n_chunks, CHUNK), jnp.int32),
            pltpu.VMEM((n_chunks, CHUNK), jnp.int32),
            pltpu.VMEM((CHUNK, D), jnp.float32),
            pltpu.VMEM((CHUNK, D), jnp.float32),
            pltpu.VMEM((CHUNK, D), jnp.float32),
            pltpu.VMEM((CHUNK, D), jnp.float32),
            pltpu.VMEM((CHUNK,), jnp.float32),
            pltpu.SemaphoreType.DMA,
            pltpu.SemaphoreType.DMA,
        ],
    )(vi_idx, vj_idx, node_t, ctx_t)


def _tc_loss_body(ip_ref, lab_ref, out_ref):
    x = lab_ref[...] * ip_ref[...]
    ls = jnp.minimum(x, 0.0) - jnp.log1p(jnp.exp(-jnp.abs(x)))
    out_ref[0, 0] = -jnp.sum(ls)


def _tc_loss(ip, labels):
    rows = ip.shape[0]
    return pl.pallas_call(
        _tc_loss_body,
        out_shape=jax.ShapeDtypeStruct((1, 1), jnp.float32),
        in_specs=[
            pl.BlockSpec((rows, D), lambda: (0, 0)),
            pl.BlockSpec((rows, D), lambda: (0, 0)),
        ],
        out_specs=pl.BlockSpec(memory_space=pltpu.SMEM),
    )(ip, labels)


def kernel(v_i, v_j, labels, batch_size, node_embeddings, contextnode_embeddings):
    n_pairs = v_i.shape[0] * v_i.shape[1]
    n_chunks = n_pairs // (NW * CHUNK)
    vi_idx = v_i.reshape(NW, n_chunks, CHUNK).astype(jnp.int32)
    vj_idx = v_j.reshape(NW, n_chunks, CHUNK).astype(jnp.int32)
    ip = _inner_products(vi_idx, vj_idx, node_embeddings, contextnode_embeddings)
    ip2 = ip.reshape(n_pairs // D, D)
    lab2 = labels.reshape(n_pairs // D, D)
    total = _tc_loss(ip2, lab2)[0, 0]
    return total / batch_size


# speedup vs baseline: 4.8834x; 1.0067x over previous
import functools

import jax
import jax.numpy as jnp
from jax import lax
from jax.experimental import pallas as pl
from jax.experimental.pallas import tpu as pltpu
from jax.experimental.pallas import tpu_sc as plsc

D = 128
NC, NS, L = 2, 16, 16
NW = NC * NS
CHUNK = 128
GROUPS = CHUNK // L


def _sc_body(vi_idx, vj_idx, node_t, ctx_t, out,
             idx_i, idx_j, vi_a, vj_a, vi_b, vj_b, out_v, sem_a, sem_b):
    n_chunks = vi_idx.shape[1]
    wid = lax.axis_index("s") * NC + lax.axis_index("c")
    pltpu.sync_copy(vi_idx.at[wid], idx_i)
    pltpu.sync_copy(vj_idx.at[wid], idx_j)

    def start(c, vi_buf, vj_buf, sem):
        pltpu.async_copy(node_t.at[idx_i.at[c]], vi_buf, sem)
        pltpu.async_copy(ctx_t.at[idx_j.at[c]], vj_buf, sem)

    def drain(vi_buf, vj_buf, sem):
        pltpu.make_async_copy(node_t.at[idx_i.at[0]], vi_buf, sem).wait()
        pltpu.make_async_copy(ctx_t.at[idx_j.at[0]], vj_buf, sem).wait()

    def compute(c, vi_rows, vj_rows):
        for g in range(GROUPS):
            rows = lax.iota(jnp.int32, L) + g * L

            def d_body(t, carry):
                a0, a1, a2, a3, dd = carry
                accs = [a0, a1, a2, a3]
                for u in range(16):
                    a = plsc.load_gather(vi_rows, [rows, dd])
                    b = plsc.load_gather(vj_rows, [rows, dd])
                    accs[u % 4] = accs[u % 4] + a * b
                    dd = jnp.bitwise_and(dd + 1, D - 1)
                return (accs[0], accs[1], accs[2], accs[3], dd)

            z = jnp.zeros((L,), jnp.float32)
            fin = lax.fori_loop(0, D // 16, d_body,
                                (z, z, z, z, lax.iota(jnp.int32, L)))
            out_v[c, pl.ds(g * L, L)] = (fin[0] + fin[1]) + (fin[2] + fin[3])

    start(0, vi_a, vj_a, sem_a)

    def super_body(t, carry):
        c0 = 2 * t
        start(c0 + 1, vi_b, vj_b, sem_b)
        drain(vi_a, vj_a, sem_a)
        compute(c0, vi_a, vj_a)

        @pl.when(t + 1 < n_chunks // 2)
        def _():
            start(c0 + 2, vi_a, vj_a, sem_a)

        drain(vi_b, vj_b, sem_b)
        compute(c0 + 1, vi_b, vj_b)
        return carry

    lax.fori_loop(0, n_chunks // 2, super_body, 0)
    pltpu.sync_copy(out_v, out.at[wid])


def _inner_products(vi_idx, vj_idx, node_t, ctx_t):
    n_chunks = vi_idx.shape[1]
    mesh = plsc.VectorSubcoreMesh(core_axis_name="c", subcore_axis_name="s")
    return pl.kernel(
        _sc_body,
        out_type=jax.ShapeDtypeStruct((NW, n_chunks, CHUNK), jnp.float32),
        mesh=mesh,
        compiler_params=pltpu.CompilerParams(needs_layout_passes=False),
        scratch_types=[
            pltpu.VMEM((n_chunks, CHUNK), jnp.int32),
            pltpu.VMEM((n_chunks, CHUNK), jnp.int32),
            pltpu.VMEM((CHUNK, D), jnp.float32),
            pltpu.VMEM((CHUNK, D), jnp.float32),
            pltpu.VMEM((CHUNK, D), jnp.float32),
            pltpu.VMEM((CHUNK, D), jnp.float32),
            pltpu.VMEM((n_chunks, CHUNK), jnp.float32),
            pltpu.SemaphoreType.DMA,
            pltpu.SemaphoreType.DMA,
        ],
    )(vi_idx, vj_idx, node_t, ctx_t)


def _tc_loss_body(ip_ref, lab_ref, out_ref):
    x = lab_ref[...] * ip_ref[...]
    ls = jnp.minimum(x, 0.0) - jnp.log1p(jnp.exp(-jnp.abs(x)))
    out_ref[0, 0] = -jnp.sum(ls)


def _tc_loss(ip, labels):
    rows = ip.shape[0]
    return pl.pallas_call(
        _tc_loss_body,
        out_shape=jax.ShapeDtypeStruct((1, 1), jnp.float32),
        in_specs=[
            pl.BlockSpec((rows, D), lambda: (0, 0)),
            pl.BlockSpec((rows, D), lambda: (0, 0)),
        ],
        out_specs=pl.BlockSpec(memory_space=pltpu.SMEM),
    )(ip, labels)


def kernel(v_i, v_j, labels, batch_size, node_embeddings, contextnode_embeddings):
    n_pairs = v_i.shape[0] * v_i.shape[1]
    n_chunks = n_pairs // (NW * CHUNK)
    vi_idx = v_i.reshape(NW, n_chunks, CHUNK).astype(jnp.int32)
    vj_idx = v_j.reshape(NW, n_chunks, CHUNK).astype(jnp.int32)
    ip = _inner_products(vi_idx, vj_idx, node_embeddings, contextnode_embeddings)
    ip2 = ip.reshape(n_pairs // D, D)
    lab2 = labels.reshape(n_pairs // D, D)
    total = _tc_loss(ip2, lab2)[0, 0]
    return total / batch_size


# speedup vs baseline: 5.0234x; 1.0287x over previous
import jax
import jax.numpy as jnp
from jax import lax
from jax.experimental import pallas as pl
from jax.experimental.pallas import tpu as pltpu
from jax.experimental.pallas import tpu_sc as plsc

D = 128
NC, NS, L = 2, 16, 16
NW = NC * NS
CH_ROWS = 16
SS_ROWS = 2 * CH_ROWS


def _sc_body(vi_idx, vj_idx, node_t, ctx_t, out,
             ia_i, ia_j, ib_i, ib_j, vi_a, vj_a, vi_b, vj_b, out_v,
             sem_a, sem_b, sem_idx):
    b, k = vi_idx.shape
    rows_w = b // NW
    chunk = CH_ROWS * k
    groups = chunk // L
    n_chunks = rows_w // CH_ROWS
    n_ss = n_chunks // 2
    wid = lax.axis_index("s") * NC + lax.axis_index("c")
    row0 = wid * rows_w

    def fire_idx(s, bi, bj):
        src_i = vi_idx.at[pl.ds(row0 + s * SS_ROWS, SS_ROWS)]
        src_j = vj_idx.at[pl.ds(row0 + s * SS_ROWS, SS_ROWS)]
        pltpu.async_copy(src_i, bi, sem_idx)
        pltpu.async_copy(src_j, bj, sem_idx)

    def drain_idx(bi, bj):
        pltpu.make_async_copy(vi_idx.at[pl.ds(0, SS_ROWS)], bi, sem_idx).wait()
        pltpu.make_async_copy(vj_idx.at[pl.ds(0, SS_ROWS)], bj, sem_idx).wait()

    def start_rows(half, bi, bj, vi_buf, vj_buf, sem):
        for q in range(CH_ROWS):
            r = half * CH_ROWS + q
            pltpu.async_copy(node_t.at[bi.at[r]], vi_buf.at[pl.ds(q * 8, k)], sem)
            pltpu.async_copy(ctx_t.at[bj.at[r]], vj_buf.at[pl.ds(q * 8, k)], sem)

    def drain_rows(vi_buf, vj_buf, sem):
        for q in range(CH_ROWS):
            pltpu.make_async_copy(node_t.at[ia_i.at[0]],
                                  vi_buf.at[pl.ds(q * 8, k)], sem).wait()
            pltpu.make_async_copy(ctx_t.at[ia_j.at[0]],
                                  vj_buf.at[pl.ds(q * 8, k)], sem).wait()

    def compute(c, vi_rows, vj_rows):
        for g in range(groups):
            p = lax.iota(jnp.int32, L) + g * L
            rows = p + (8 - k) * jnp.right_shift(p * 10923, 16)

            def d_body(t, carry):
                a0, a1, a2, a3, dd = carry
                accs = [a0, a1, a2, a3]
                for u in range(16):
                    a = plsc.load_gather(vi_rows, [rows, dd])
                    b2 = plsc.load_gather(vj_rows, [rows, dd])
                    accs[u % 4] = accs[u % 4] + a * b2
                    dd = jnp.bitwise_and(dd + 1, D - 1)
                return (accs[0], accs[1], accs[2], accs[3], dd)

            z = jnp.zeros((L,), jnp.float32)
            fin = lax.fori_loop(0, D // 16, d_body,
                                (z, z, z, z, lax.iota(jnp.int32, L)))
            out_v[c, pl.ds(g * L, L)] = (fin[0] + fin[1]) + (fin[2] + fin[3])

    def superstep(s, iX_i, iX_j, iY_i, iY_j):
        c0 = 2 * s
        start_rows(1, iX_i, iX_j, vi_b, vj_b, sem_b)
        drain_rows(vi_a, vj_a, sem_a)
        compute(c0, vi_a, vj_a)

        @pl.when(s + 1 < n_ss)
        def _():
            drain_idx(iY_i, iY_j)
            start_rows(0, iY_i, iY_j, vi_a, vj_a, sem_a)

        drain_rows(vi_b, vj_b, sem_b)

        @pl.when(s + 2 < n_ss)
        def _():
            fire_idx(s + 2, iX_i, iX_j)

        compute(c0 + 1, vi_b, vj_b)

    pltpu.sync_copy(vi_idx.at[pl.ds(row0, SS_ROWS)], ia_i)
    pltpu.sync_copy(vj_idx.at[pl.ds(row0, SS_ROWS)], ia_j)
    fire_idx(1, ib_i, ib_j)
    start_rows(0, ia_i, ia_j, vi_a, vj_a, sem_a)

    def pair_body(u, carry):
        superstep(2 * u, ia_i, ia_j, ib_i, ib_j)
        superstep(2 * u + 1, ib_i, ib_j, ia_i, ia_j)
        return carry

    lax.fori_loop(0, n_ss // 2, pair_body, 0)
    pltpu.sync_copy(out_v, out.at[wid])


def _inner_products(vi_idx, vj_idx, node_t, ctx_t):
    b, k = vi_idx.shape
    rows_w = b // NW
    chunk = CH_ROWS * k
    n_chunks = rows_w // CH_ROWS
    mesh = plsc.VectorSubcoreMesh(core_axis_name="c", subcore_axis_name="s")
    return pl.kernel(
        _sc_body,
        out_type=jax.ShapeDtypeStruct((NW, n_chunks, chunk), jnp.float32),
        mesh=mesh,
        compiler_params=pltpu.CompilerParams(needs_layout_passes=False),
        scratch_types=[
            pltpu.VMEM((SS_ROWS, k), jnp.int32),
            pltpu.VMEM((SS_ROWS, k), jnp.int32),
            pltpu.VMEM((SS_ROWS, k), jnp.int32),
            pltpu.VMEM((SS_ROWS, k), jnp.int32),
            pltpu.VMEM((CH_ROWS * 8, D), jnp.float32),
            pltpu.VMEM((CH_ROWS * 8, D), jnp.float32),
            pltpu.VMEM((CH_ROWS * 8, D), jnp.float32),
            pltpu.VMEM((CH_ROWS * 8, D), jnp.float32),
            pltpu.VMEM((n_chunks, chunk), jnp.float32),
            pltpu.SemaphoreType.DMA,
            pltpu.SemaphoreType.DMA,
            pltpu.SemaphoreType.DMA,
        ],
    )(vi_idx, vj_idx, node_t, ctx_t)


def _tc_loss_body(ip_ref, lab_ref, out_ref):
    x = lab_ref[...] * ip_ref[...]
    ls = jnp.minimum(x, 0.0) - jnp.log1p(jnp.exp(-jnp.abs(x)))
    out_ref[0, 0] = -jnp.sum(ls)


def _tc_loss(ip, labels):
    rows = ip.shape[0]
    return pl.pallas_call(
        _tc_loss_body,
        out_shape=jax.ShapeDtypeStruct((1, 1), jnp.float32),
        in_specs=[
            pl.BlockSpec((rows, D), lambda: (0, 0)),
            pl.BlockSpec((rows, D), lambda: (0, 0)),
        ],
        out_specs=pl.BlockSpec(memory_space=pltpu.SMEM),
    )(ip, labels)


def kernel(v_i, v_j, labels, batch_size, node_embeddings, contextnode_embeddings):
    n_pairs = v_i.shape[0] * v_i.shape[1]
    ip = _inner_products(v_i.astype(jnp.int32), v_j.astype(jnp.int32),
                         node_embeddings, contextnode_embeddings)
    ip2 = ip.reshape(n_pairs // D, D)
    lab2 = labels.reshape(n_pairs // D, D)
    total = _tc_loss(ip2, lab2)[0, 0]
    return total / batch_size


# speedup vs baseline: 5.1402x; 1.0232x over previous
import jax
import jax.numpy as jnp
from jax import lax
from jax.experimental import pallas as pl
from jax.experimental.pallas import tpu as pltpu
from jax.experimental.pallas import tpu_sc as plsc

D = 128
NC, NS, L = 2, 16, 16
NW = NC * NS
CH_ROWS = 16
SS_ROWS = 2 * CH_ROWS


def _sc_body(vi_idx, vj_idx, node_t, ctx_t, out,
             ia_i, ia_j, ib_i, ib_j, ci_i, ci_j, vi_a, vj_a, vi_b, vj_b,
             out_v, sem_a, sem_b, sem_idx):
    b, k = vi_idx.shape
    rows_w = b // NW
    chunk = CH_ROWS * k
    groups = chunk // L
    n_chunks = rows_w // CH_ROWS
    n_ss = n_chunks // 2
    wid = lax.axis_index("s") * NC + lax.axis_index("c")
    row0 = wid * rows_w

    def fire_idx(s, bi, bj):
        src_i = vi_idx.at[pl.ds(row0 + s * SS_ROWS, SS_ROWS)]
        src_j = vj_idx.at[pl.ds(row0 + s * SS_ROWS, SS_ROWS)]
        pltpu.async_copy(src_i, bi, sem_idx)
        pltpu.async_copy(src_j, bj, sem_idx)

    def drain_idx(bi, bj):
        pltpu.make_async_copy(vi_idx.at[pl.ds(0, SS_ROWS)], bi, sem_idx).wait()
        pltpu.make_async_copy(vj_idx.at[pl.ds(0, SS_ROWS)], bj, sem_idx).wait()

    def compact(half, bi, bj, par):
        for g in range(groups):
            p = lax.iota(jnp.int32, L) + g * L
            qv = jnp.right_shift(p * 10923, 16)
            jv = p - qv * k
            qv = qv + half * CH_ROWS
            ci_i[par, pl.ds(g * L, L)] = plsc.load_gather(bi, [qv, jv])
            ci_j[par, pl.ds(g * L, L)] = plsc.load_gather(bj, [qv, jv])

    def start_rows(par, vi_buf, vj_buf, sem):
        pltpu.async_copy(node_t.at[ci_i.at[par]], vi_buf, sem)
        pltpu.async_copy(ctx_t.at[ci_j.at[par]], vj_buf, sem)

    def drain_rows(vi_buf, vj_buf, sem):
        pltpu.make_async_copy(node_t.at[ci_i.at[0]], vi_buf, sem).wait()
        pltpu.make_async_copy(ctx_t.at[ci_j.at[0]], vj_buf, sem).wait()

    def compute(c, vi_rows, vj_rows):
        for g in range(groups):
            rows = lax.iota(jnp.int32, L) + g * L

            def d_body(t, carry):
                a0, a1, a2, a3, dd = carry
                accs = [a0, a1, a2, a3]
                for u in range(16):
                    a = plsc.load_gather(vi_rows, [rows, dd])
                    b2 = plsc.load_gather(vj_rows, [rows, dd])
                    accs[u % 4] = accs[u % 4] + a * b2
                    dd = jnp.bitwise_and(dd + 1, D - 1)
                return (accs[0], accs[1], accs[2], accs[3], dd)

            z = jnp.zeros((L,), jnp.float32)
            fin = lax.fori_loop(0, D // 16, d_body,
                                (z, z, z, z, lax.iota(jnp.int32, L)))
            out_v[c, pl.ds(g * L, L)] = (fin[0] + fin[1]) + (fin[2] + fin[3])

    def superstep(s, iX_i, iX_j, iY_i, iY_j):
        c0 = 2 * s
        compact(1, iX_i, iX_j, 1)
        start_rows(1, vi_b, vj_b, sem_b)
        drain_rows(vi_a, vj_a, sem_a)
        compute(c0, vi_a, vj_a)

        @pl.when(s + 1 < n_ss)
        def _():
            drain_idx(iY_i, iY_j)
            compact(0, iY_i, iY_j, 0)
            start_rows(0, vi_a, vj_a, sem_a)

        drain_rows(vi_b, vj_b, sem_b)

        @pl.when(s + 2 < n_ss)
        def _():
            fire_idx(s + 2, iX_i, iX_j)

        compute(c0 + 1, vi_b, vj_b)

    pltpu.sync_copy(vi_idx.at[pl.ds(row0, SS_ROWS)], ia_i)
    pltpu.sync_copy(vj_idx.at[pl.ds(row0, SS_ROWS)], ia_j)
    fire_idx(1, ib_i, ib_j)
    compact(0, ia_i, ia_j, 0)
    start_rows(0, vi_a, vj_a, sem_a)

    def pair_body(u, carry):
        superstep(2 * u, ia_i, ia_j, ib_i, ib_j)
        superstep(2 * u + 1, ib_i, ib_j, ia_i, ia_j)
        return carry

    lax.fori_loop(0, n_ss // 2, pair_body, 0)
    pltpu.sync_copy(out_v, out.at[wid])


def _inner_products(vi_idx, vj_idx, node_t, ctx_t):
    b, k = vi_idx.shape
    rows_w = b // NW
    chunk = CH_ROWS * k
    n_chunks = rows_w // CH_ROWS
    mesh = plsc.VectorSubcoreMesh(core_axis_name="c", subcore_axis_name="s")
    return pl.kernel(
        _sc_body,
        out_type=jax.ShapeDtypeStruct((NW, n_chunks, chunk), jnp.float32),
        mesh=mesh,
        compiler_params=pltpu.CompilerParams(needs_layout_passes=False),
        scratch_types=[
            pltpu.VMEM((SS_ROWS, k), jnp.int32),
            pltpu.VMEM((SS_ROWS, k), jnp.int32),
            pltpu.VMEM((SS_ROWS, k), jnp.int32),
            pltpu.VMEM((SS_ROWS, k), jnp.int32),
            pltpu.VMEM((2, chunk), jnp.int32),
            pltpu.VMEM((2, chunk), jnp.int32),
            pltpu.VMEM((chunk, D), jnp.float32),
            pltpu.VMEM((chunk, D), jnp.float32),
            pltpu.VMEM((chunk, D), jnp.float32),
            pltpu.VMEM((chunk, D), jnp.float32),
            pltpu.VMEM((n_chunks, chunk), jnp.float32),
            pltpu.SemaphoreType.DMA,
            pltpu.SemaphoreType.DMA,
            pltpu.SemaphoreType.DMA,
        ],
    )(vi_idx, vj_idx, node_t, ctx_t)


def _tc_loss_body(ip_ref, lab_ref, out_ref):
    x = lab_ref[...] * ip_ref[...]
    ls = jnp.minimum(x, 0.0) - jnp.log1p(jnp.exp(-jnp.abs(x)))
    out_ref[0, 0] = -jnp.sum(ls)


def _tc_loss(ip, labels):
    rows = ip.shape[0]
    return pl.pallas_call(
        _tc_loss_body,
        out_shape=jax.ShapeDtypeStruct((1, 1), jnp.float32),
        in_specs=[
            pl.BlockSpec((rows, D), lambda: (0, 0)),
            pl.BlockSpec((rows, D), lambda: (0, 0)),
        ],
        out_specs=pl.BlockSpec(memory_space=pltpu.SMEM),
    )(ip, labels)


def kernel(v_i, v_j, labels, batch_size, node_embeddings, contextnode_embeddings):
    n_pairs = v_i.shape[0] * v_i.shape[1]
    ip = _inner_products(v_i.astype(jnp.int32), v_j.astype(jnp.int32),
                         node_embeddings, contextnode_embeddings)
    ip2 = ip.reshape(n_pairs // D, D)
    lab2 = labels.reshape(n_pairs // D, D)
    total = _tc_loss(ip2, lab2)[0, 0]
    return total / batch_size


# speedup vs baseline: 5.2524x; 1.0218x over previous
import jax
import jax.numpy as jnp
from jax import lax
from jax.experimental import pallas as pl
from jax.experimental.pallas import tpu as pltpu
from jax.experimental.pallas import tpu_sc as plsc

D = 128
NC, NS, L = 2, 16, 16
NW = NC * NS
CH_ROWS = 16
SS_ROWS = 2 * CH_ROWS


def _sc_body(vi_idx, vj_idx, node_t, ctx_t, out,
             ia_i, ia_j, ib_i, ib_j, ci_i, ci_j, vi_a, vj_a, vi_b, vj_b,
             out_v, sem_a, sem_b, sem_idx):
    b, k = vi_idx.shape
    rows_w = b // NW
    chunk = CH_ROWS * k
    groups = chunk // L
    n_chunks = rows_w // CH_ROWS
    n_ss = n_chunks // 2
    wid = lax.axis_index("s") * NC + lax.axis_index("c")
    row0 = wid * rows_w

    def fire_idx(s, bi, bj):
        src_i = vi_idx.at[pl.ds(row0 + s * SS_ROWS, SS_ROWS)]
        src_j = vj_idx.at[pl.ds(row0 + s * SS_ROWS, SS_ROWS)]
        pltpu.async_copy(src_i, bi, sem_idx)
        pltpu.async_copy(src_j, bj, sem_idx)

    def drain_idx(bi, bj):
        pltpu.make_async_copy(vi_idx.at[pl.ds(0, SS_ROWS)], bi, sem_idx).wait()
        pltpu.make_async_copy(vj_idx.at[pl.ds(0, SS_ROWS)], bj, sem_idx).wait()

    def compact(half, bi, bj, par):
        for g in range(groups):
            p = lax.iota(jnp.int32, L) + g * L
            qv = jnp.right_shift(p * 10923, 16)
            jv = p - qv * k
            qv = qv + half * CH_ROWS
            ci_i[par, pl.ds(g * L, L)] = plsc.load_gather(bi, [qv, jv])
            ci_j[par, pl.ds(g * L, L)] = plsc.load_gather(bj, [qv, jv])

    def start_rows(par, vi_buf, vj_buf, sem):
        pltpu.async_copy(node_t.at[ci_i.at[par]], vi_buf, sem)
        pltpu.async_copy(ctx_t.at[ci_j.at[par]], vj_buf, sem)

    def drain_rows(vi_buf, vj_buf, sem):
        pltpu.make_async_copy(node_t.at[ci_i.at[0]], vi_buf, sem).wait()
        pltpu.make_async_copy(ctx_t.at[ci_j.at[0]], vj_buf, sem).wait()

    def compute(c, vi_rows, vj_rows):
        for g in range(groups):
            rows = lax.iota(jnp.int32, L) + g * L

            def d_body(t, carry):
                a0, a1, a2, a3, dd = carry
                accs = [a0, a1, a2, a3]
                for u in range(16):
                    a = plsc.load_gather(vi_rows, [rows, dd])
                    b2 = plsc.load_gather(vj_rows, [rows, dd])
                    accs[u % 4] = accs[u % 4] + a * b2
                    dd = jnp.bitwise_and(dd + 1, D - 1)
                return (accs[0], accs[1], accs[2], accs[3], dd)

            z = jnp.zeros((L,), jnp.float32)
            fin = lax.fori_loop(0, D // 16, d_body,
                                (z, z, z, z, lax.iota(jnp.int32, L)))
            sidx = c * (chunk // L) + g
            out_v[sidx // 8, pl.ds((sidx % 8) * L, L)] = (
                (fin[0] + fin[1]) + (fin[2] + fin[3]))

    def superstep(s, iX_i, iX_j, iY_i, iY_j):
        c0 = 2 * s
        compact(1, iX_i, iX_j, 1)
        start_rows(1, vi_b, vj_b, sem_b)
        drain_rows(vi_a, vj_a, sem_a)
        compute(c0, vi_a, vj_a)

        @pl.when(s + 1 < n_ss)
        def _():
            drain_idx(iY_i, iY_j)
            compact(0, iY_i, iY_j, 0)
            start_rows(0, vi_a, vj_a, sem_a)

        drain_rows(vi_b, vj_b, sem_b)

        @pl.when(s + 2 < n_ss)
        def _():
            fire_idx(s + 2, iX_i, iX_j)

        compute(c0 + 1, vi_b, vj_b)

    pltpu.sync_copy(vi_idx.at[pl.ds(row0, SS_ROWS)], ia_i)
    pltpu.sync_copy(vj_idx.at[pl.ds(row0, SS_ROWS)], ia_j)
    fire_idx(1, ib_i, ib_j)
    compact(0, ia_i, ia_j, 0)
    start_rows(0, vi_a, vj_a, sem_a)

    def pair_body(u, carry):
        superstep(2 * u, ia_i, ia_j, ib_i, ib_j)
        superstep(2 * u + 1, ib_i, ib_j, ia_i, ia_j)
        return carry

    lax.fori_loop(0, n_ss // 2, pair_body, 0)
    rows_out = (rows_w * k) // D
    pltpu.sync_copy(out_v, out.at[pl.ds(wid * rows_out, rows_out)])


def _inner_products(vi_idx, vj_idx, node_t, ctx_t):
    b, k = vi_idx.shape
    rows_w = b // NW
    chunk = CH_ROWS * k
    n_chunks = rows_w // CH_ROWS
    mesh = plsc.VectorSubcoreMesh(core_axis_name="c", subcore_axis_name="s")
    return pl.kernel(
        _sc_body,
        out_type=jax.ShapeDtypeStruct((b * k // D, D), jnp.float32),
        mesh=mesh,
        compiler_params=pltpu.CompilerParams(needs_layout_passes=False),
        scratch_types=[
            pltpu.VMEM((SS_ROWS, k), jnp.int32),
            pltpu.VMEM((SS_ROWS, k), jnp.int32),
            pltpu.VMEM((SS_ROWS, k), jnp.int32),
            pltpu.VMEM((SS_ROWS, k), jnp.int32),
            pltpu.VMEM((2, chunk), jnp.int32),
            pltpu.VMEM((2, chunk), jnp.int32),
            pltpu.VMEM((chunk, D), jnp.float32),
            pltpu.VMEM((chunk, D), jnp.float32),
            pltpu.VMEM((chunk, D), jnp.float32),
            pltpu.VMEM((chunk, D), jnp.float32),
            pltpu.VMEM((rows_w * k // D, D), jnp.float32),
            pltpu.SemaphoreType.DMA,
            pltpu.SemaphoreType.DMA,
            pltpu.SemaphoreType.DMA,
        ],
    )(vi_idx, vj_idx, node_t, ctx_t)


def _tc_loss_body(ip_ref, lab_ref, out_ref):
    x = lab_ref[...] * ip_ref[...]
    ls = jnp.minimum(x, 0.0) - jnp.log1p(jnp.exp(-jnp.abs(x)))
    out_ref[0, 0] = -jnp.sum(ls)


def _tc_loss(ip, labels):
    rows = ip.shape[0]
    return pl.pallas_call(
        _tc_loss_body,
        out_shape=jax.ShapeDtypeStruct((1, 1), jnp.float32),
        in_specs=[
            pl.BlockSpec((rows, D), lambda: (0, 0)),
            pl.BlockSpec((rows, D), lambda: (0, 0)),
        ],
        out_specs=pl.BlockSpec(memory_space=pltpu.SMEM),
    )(ip, labels)


def kernel(v_i, v_j, labels, batch_size, node_embeddings, contextnode_embeddings):
    n_pairs = v_i.shape[0] * v_i.shape[1]
    ip2 = _inner_products(v_i.astype(jnp.int32), v_j.astype(jnp.int32),
                          node_embeddings, contextnode_embeddings)
    lab2 = labels.reshape(n_pairs // D, D)
    total = _tc_loss(ip2, lab2)[0, 0]
    return total / batch_size
